# baseline (speedup 1.0000x reference)
import functools

import jax
import jax.numpy as jnp
from jax import lax
from jax.experimental import pallas as pl
from jax.experimental.pallas import tpu as pltpu
from jax.experimental.pallas import tpu_sc as plsc

BATCH = 16384
EMBED = 64
PAIR = 2 * EMBED
NUM_CORES = 2
NUM_SUBCORES = 16
NUM_TILES = NUM_CORES * NUM_SUBCORES
B_PER_TILE = BATCH // NUM_TILES
ROUND_ROWS = 128
N_ROUNDS = B_PER_TILE // ROUND_ROWS
LANES = 16
DEPAD_BLOCK = 10000


def _repack(table):
    rows = table.shape[0]
    n_blocks = rows // DEPAD_BLOCK

    def body(x_ref, o_ref):
        x = x_ref[...].reshape(DEPAD_BLOCK // 2, 2, EMBED)
        o_ref[...] = jnp.concatenate([x[:, 0, :], x[:, 1, :]], axis=1)

    return pl.pallas_call(
        body,
        grid=(n_blocks,),
        in_specs=[pl.BlockSpec((DEPAD_BLOCK, EMBED), lambda i: (i, 0))],
        out_specs=pl.BlockSpec((DEPAD_BLOCK // 2, PAIR), lambda i: (i, 0)),
        out_shape=jax.ShapeDtypeStruct((rows // 2, PAIR), jnp.float32),
        compiler_params=pltpu.CompilerParams(
            dimension_semantics=("parallel",)),
    )(table)


def _sc_kernel(ut_pair, at_pair, user_ids, ad_ids, w_vec, b_vec):
    mesh = plsc.VectorSubcoreMesh(
        core_axis_name="c",
        subcore_axis_name="s",
        num_cores=NUM_CORES,
        num_subcores=NUM_SUBCORES,
    )

    cp = pltpu.CompilerParams(needs_layout_passes=False)

    @functools.partial(
        pl.kernel,
        out_type=jax.ShapeDtypeStruct((BATCH,), jnp.float32),
        mesh=mesh,
        compiler_params=cp,
        scratch_types=[
            pltpu.VMEM((B_PER_TILE,), jnp.int32),
            pltpu.VMEM((B_PER_TILE,), jnp.int32),
            pltpu.VMEM((B_PER_TILE,), jnp.int32),
            pltpu.VMEM((B_PER_TILE,), jnp.int32),
            pltpu.VMEM((ROUND_ROWS, PAIR), jnp.float32),
            pltpu.VMEM((ROUND_ROWS, PAIR), jnp.float32),
            pltpu.VMEM((B_PER_TILE,), jnp.float32),
            pltpu.VMEM((LANES,), jnp.float32),
            pltpu.VMEM((LANES,), jnp.float32),
            pltpu.SemaphoreType.DMA,
        ],
    )
    def kern(ut_hbm, at_hbm, uid_hbm, aid_hbm, w_hbm, b_hbm, out_hbm,
             uidx_v, aidx_v, upair_v, apair_v, ubuf_v, abuf_v,
             out_v, w_v, b_v, sem):
        tile = lax.axis_index("s") * NUM_CORES + lax.axis_index("c")
        base = tile * B_PER_TILE

        pltpu.sync_copy(uid_hbm.at[pl.ds(base, B_PER_TILE)], uidx_v)
        pltpu.sync_copy(aid_hbm.at[pl.ds(base, B_PER_TILE)], aidx_v)
        pltpu.sync_copy(w_hbm, w_v)
        pltpu.sync_copy(b_hbm, b_v)

        @pl.loop(0, B_PER_TILE, step=LANES)
        def _(k):
            sl = pl.ds(k, LANES)
            upair_v[sl] = lax.shift_right_logical(uidx_v[sl], 1)
            apair_v[sl] = lax.shift_right_logical(aidx_v[sl], 1)

        w = w_v[...]
        b = b_v[...]
        lane = lax.iota(jnp.int32, LANES)

        for rd in range(N_ROUNDS):
            src = pl.ds(rd * ROUND_ROWS, ROUND_ROWS)
            cu = pltpu.async_copy(ut_hbm.at[upair_v.at[src]], ubuf_v, sem)
            ca = pltpu.async_copy(at_hbm.at[apair_v.at[src]], abuf_v, sem)
            cu.wait()
            ca.wait()

            @pl.loop(0, ROUND_ROWS, step=LANES)
            def _(g):
                off = rd * ROUND_ROWS + g
                sl = pl.ds(off, LANES)
                uhalf = (uidx_v[sl] & 1) * EMBED
                ahalf = (aidx_v[sl] & 1) * EMBED
                rows = lane + g
                acc = jnp.zeros((LANES,), jnp.float32)
                for j in range(EMBED):
                    rot = (lane + j) & (EMBED - 1)
                    u = plsc.load_gather(ubuf_v, [rows, uhalf + rot])
                    a = plsc.load_gather(abuf_v, [rows, ahalf + rot])
                    acc = acc + u * a
                z = acc * w + b
                out_v[sl] = 1.0 / (1.0 + jnp.exp(-z))

        pltpu.sync_copy(out_v, out_hbm.at[pl.ds(base, B_PER_TILE)])

    return kern(ut_pair, at_pair, user_ids, ad_ids, w_vec, b_vec)


@jax.jit
def kernel(user_ids, ad_ids, user_table, ad_table, fc_w, fc_b):
    ut_pair = _repack(user_table)
    at_pair = _repack(ad_table)
    w_vec = jnp.broadcast_to(fc_w.reshape(()), (LANES,)).astype(jnp.float32)
    b_vec = jnp.broadcast_to(fc_b.reshape(()), (LANES,)).astype(jnp.float32)
    out = _sc_kernel(ut_pair, at_pair,
                     user_ids.astype(jnp.int32), ad_ids.astype(jnp.int32),
                     w_vec, b_vec)
    return out.reshape(BATCH, 1)


# speedup vs baseline: 1.0636x; 1.0636x over previous
import functools

import jax
import jax.numpy as jnp
from jax import lax
from jax.experimental import pallas as pl
from jax.experimental.pallas import tpu as pltpu
from jax.experimental.pallas import tpu_sc as plsc

BATCH = 16384
EMBED = 64
WIDE = 128
NUM_CORES = 2
NUM_SUBCORES = 16
NUM_TILES = NUM_CORES * NUM_SUBCORES
B_PER_TILE = BATCH // NUM_TILES
ROUND_ROWS = 128
N_ROUNDS = B_PER_TILE // ROUND_ROWS
LANES = 16
WIDEN_BLOCK = 10000


def _widen(table):
    rows = table.shape[0]
    n_blocks = rows // WIDEN_BLOCK

    def body(x_ref, o_ref):
        o_ref[:, 0:EMBED] = x_ref[...]

    return pl.pallas_call(
        body,
        grid=(n_blocks,),
        in_specs=[pl.BlockSpec((WIDEN_BLOCK, EMBED), lambda i: (i, 0))],
        out_specs=pl.BlockSpec((WIDEN_BLOCK, WIDE), lambda i: (i, 0)),
        out_shape=jax.ShapeDtypeStruct((rows, WIDE), jnp.float32),
        compiler_params=pltpu.CompilerParams(
            dimension_semantics=("parallel",)),
    )(table)


def _sc_kernel(ut_wide, at_wide, user_ids, ad_ids, w_vec, b_vec):
    mesh = plsc.VectorSubcoreMesh(
        core_axis_name="c",
        subcore_axis_name="s",
        num_cores=NUM_CORES,
        num_subcores=NUM_SUBCORES,
    )

    cp = pltpu.CompilerParams(needs_layout_passes=False)

    @functools.partial(
        pl.kernel,
        out_type=jax.ShapeDtypeStruct((BATCH,), jnp.float32),
        mesh=mesh,
        compiler_params=cp,
        scratch_types=[
            pltpu.VMEM((B_PER_TILE,), jnp.int32),
            pltpu.VMEM((B_PER_TILE,), jnp.int32),
            pltpu.VMEM((ROUND_ROWS, WIDE), jnp.float32),
            pltpu.VMEM((ROUND_ROWS, WIDE), jnp.float32),
            pltpu.VMEM((B_PER_TILE,), jnp.float32),
            pltpu.VMEM((LANES,), jnp.float32),
            pltpu.VMEM((LANES,), jnp.float32),
            pltpu.SemaphoreType.DMA,
        ],
    )
    def kern(ut_hbm, at_hbm, uid_hbm, aid_hbm, w_hbm, b_hbm, out_hbm,
             uidx_v, aidx_v, ubuf_v, abuf_v, out_v, w_v, b_v, sem):
        tile = lax.axis_index("s") * NUM_CORES + lax.axis_index("c")
        base = tile * B_PER_TILE

        pltpu.sync_copy(uid_hbm.at[pl.ds(base, B_PER_TILE)], uidx_v)
        pltpu.sync_copy(aid_hbm.at[pl.ds(base, B_PER_TILE)], aidx_v)
        pltpu.sync_copy(w_hbm, w_v)
        pltpu.sync_copy(b_hbm, b_v)

        w = w_v[...]
        b = b_v[...]
        lane = lax.iota(jnp.int32, LANES)

        for rd in range(N_ROUNDS):
            src = pl.ds(rd * ROUND_ROWS, ROUND_ROWS)
            cu = pltpu.async_copy(ut_hbm.at[uidx_v.at[src]], ubuf_v, sem)
            ca = pltpu.async_copy(at_hbm.at[aidx_v.at[src]], abuf_v, sem)
            cu.wait()
            ca.wait()

            @pl.loop(0, ROUND_ROWS, step=LANES)
            def _(g):
                rows = lane + g
                acc = jnp.zeros((LANES,), jnp.float32)
                for j in range(EMBED):
                    cols = (lane + j) & (EMBED - 1)
                    u = plsc.load_gather(ubuf_v, [rows, cols])
                    a = plsc.load_gather(abuf_v, [rows, cols])
                    acc = acc + u * a
                z = acc * w + b
                out_v[pl.ds(rd * ROUND_ROWS + g, LANES)] = (
                    1.0 / (1.0 + jnp.exp(-z)))

        pltpu.sync_copy(out_v, out_hbm.at[pl.ds(base, B_PER_TILE)])

    return kern(ut_wide, at_wide, user_ids, ad_ids, w_vec, b_vec)


@jax.jit
def kernel(user_ids, ad_ids, user_table, ad_table, fc_w, fc_b):
    ut_wide = _widen(user_table)
    at_wide = _widen(ad_table)
    w_vec = jnp.broadcast_to(fc_w.reshape(()), (LANES,)).astype(jnp.float32)
    b_vec = jnp.broadcast_to(fc_b.reshape(()), (LANES,)).astype(jnp.float32)
    out = _sc_kernel(ut_wide, at_wide,
                     user_ids.astype(jnp.int32), ad_ids.astype(jnp.int32),
                     w_vec, b_vec)
    return out.reshape(BATCH, 1)


# speedup vs baseline: 1.0644x; 1.0008x over previous
import functools

import jax
import jax.numpy as jnp
from jax import lax
from jax.experimental import pallas as pl
from jax.experimental.pallas import tpu as pltpu
from jax.experimental.pallas import tpu_sc as plsc

BATCH = 16384
EMBED = 64
WIDE = 128
NUM_CORES = 2
NUM_SUBCORES = 16
NUM_TILES = NUM_CORES * NUM_SUBCORES
B_PER_TILE = BATCH // NUM_TILES
ROUND_ROWS = 128
N_ROUNDS = B_PER_TILE // ROUND_ROWS
LANES = 16
WIDEN_BLOCK = 25000


def _widen(table):
    rows = table.shape[0]
    n_blocks = rows // WIDEN_BLOCK

    def body(x_ref, o_ref):
        o_ref[:, 0:EMBED] = x_ref[...]

    return pl.pallas_call(
        body,
        grid=(n_blocks,),
        in_specs=[pl.BlockSpec((WIDEN_BLOCK, EMBED), lambda i: (i, 0))],
        out_specs=pl.BlockSpec((WIDEN_BLOCK, WIDE), lambda i: (i, 0)),
        out_shape=jax.ShapeDtypeStruct((rows, WIDE), jnp.float32),
        compiler_params=pltpu.CompilerParams(
            dimension_semantics=("parallel",)),
    )(table)


def _sc_kernel(ut_wide, at_wide, user_ids, ad_ids, w_vec, b_vec):
    mesh = plsc.VectorSubcoreMesh(
        core_axis_name="c",
        subcore_axis_name="s",
        num_cores=NUM_CORES,
        num_subcores=NUM_SUBCORES,
    )

    cp = pltpu.CompilerParams(needs_layout_passes=False)

    @functools.partial(
        pl.kernel,
        out_type=jax.ShapeDtypeStruct((BATCH,), jnp.float32),
        mesh=mesh,
        compiler_params=cp,
        scratch_types=[
            pltpu.VMEM((B_PER_TILE,), jnp.int32),
            pltpu.VMEM((B_PER_TILE,), jnp.int32),
            pltpu.VMEM((ROUND_ROWS, WIDE), jnp.float32),
            pltpu.VMEM((ROUND_ROWS, WIDE), jnp.float32),
            pltpu.VMEM((B_PER_TILE,), jnp.float32),
            pltpu.VMEM((LANES,), jnp.float32),
            pltpu.VMEM((LANES,), jnp.float32),
            pltpu.SemaphoreType.DMA,
        ],
    )
    def kern(ut_hbm, at_hbm, uid_hbm, aid_hbm, w_hbm, b_hbm, out_hbm,
             uidx_v, aidx_v, ubuf_v, abuf_v, out_v, w_v, b_v, sem):
        tile = lax.axis_index("s") * NUM_CORES + lax.axis_index("c")
        base = tile * B_PER_TILE

        pltpu.sync_copy(uid_hbm.at[pl.ds(base, B_PER_TILE)], uidx_v)
        pltpu.sync_copy(aid_hbm.at[pl.ds(base, B_PER_TILE)], aidx_v)
        pltpu.sync_copy(w_hbm, w_v)
        pltpu.sync_copy(b_hbm, b_v)

        w = w_v[...]
        b = b_v[...]
        lane = lax.iota(jnp.int32, LANES)

        for rd in range(N_ROUNDS):
            src = pl.ds(rd * ROUND_ROWS, ROUND_ROWS)
            cu = pltpu.async_copy(ut_hbm.at[uidx_v.at[src]], ubuf_v, sem)
            ca = pltpu.async_copy(at_hbm.at[aidx_v.at[src]], abuf_v, sem)
            cu.wait()
            ca.wait()

            @pl.loop(0, ROUND_ROWS, step=LANES)
            def _(g):
                rows = lane + g
                acc = jnp.zeros((LANES,), jnp.float32)
                for j in range(EMBED):
                    cols = (lane + j) & (EMBED - 1)
                    u = plsc.load_gather(ubuf_v, [rows, cols])
                    a = plsc.load_gather(abuf_v, [rows, cols])
                    acc = acc + u * a
                z = acc * w + b
                out_v[pl.ds(rd * ROUND_ROWS + g, LANES)] = (
                    1.0 / (1.0 + jnp.exp(-z)))

        pltpu.sync_copy(out_v, out_hbm.at[pl.ds(base, B_PER_TILE)])

    return kern(ut_wide, at_wide, user_ids, ad_ids, w_vec, b_vec)


@jax.jit
def kernel(user_ids, ad_ids, user_table, ad_table, fc_w, fc_b):
    ut_wide = _widen(user_table)
    at_wide = _widen(ad_table)
    w_vec = jnp.broadcast_to(fc_w.reshape(()), (LANES,)).astype(jnp.float32)
    b_vec = jnp.broadcast_to(fc_b.reshape(()), (LANES,)).astype(jnp.float32)
    out = _sc_kernel(ut_wide, at_wide,
                     user_ids.astype(jnp.int32), ad_ids.astype(jnp.int32),
                     w_vec, b_vec)
    return out.reshape(BATCH, 1)


# speedup vs baseline: 1.9821x; 1.8621x over previous
import functools

import jax
import jax.numpy as jnp
from jax import lax
from jax.experimental import pallas as pl
from jax.experimental.pallas import tpu as pltpu
from jax.experimental.pallas import tpu_sc as plsc

BATCH = 16384
EMBED = 64
NUM_CORES = 2
NUM_SUBCORES = 16
NUM_TILES = NUM_CORES * NUM_SUBCORES
B_PER_TILE = BATCH // NUM_TILES
PASS_ROWS = 32
N_PASS = B_PER_TILE // PASS_ROWS
LANES = 16


def _sc_kernel(user_table, ad_table, user_ids, ad_ids, w_vec, b_vec):
    mesh = plsc.VectorSubcoreMesh(
        core_axis_name="c",
        subcore_axis_name="s",
        num_cores=NUM_CORES,
        num_subcores=NUM_SUBCORES,
    )

    cp = pltpu.CompilerParams(
        needs_layout_passes=False, disable_bounds_checks=True)

    @functools.partial(
        pl.kernel,
        out_type=jax.ShapeDtypeStruct((BATCH,), jnp.float32),
        mesh=mesh,
        compiler_params=cp,
        scratch_types=[
            pltpu.VMEM((B_PER_TILE,), jnp.int32),
            pltpu.VMEM((B_PER_TILE,), jnp.int32),
            pltpu.VMEM((2, PASS_ROWS, EMBED), jnp.float32),
            pltpu.VMEM((2, PASS_ROWS, EMBED), jnp.float32),
            pltpu.VMEM((B_PER_TILE,), jnp.float32),
            pltpu.VMEM((LANES,), jnp.float32),
            pltpu.VMEM((LANES,), jnp.float32),
            pltpu.SemaphoreType.DMA,
            pltpu.SemaphoreType.DMA,
        ],
    )
    def kern(ut_hbm, at_hbm, uid_hbm, aid_hbm, w_hbm, b_hbm, out_hbm,
             uidx_v, aidx_v, ubuf_v, abuf_v, out_v, w_v, b_v, sem0, sem1):
        tile = lax.axis_index("s") * NUM_CORES + lax.axis_index("c")
        base = tile * B_PER_TILE

        pltpu.sync_copy(uid_hbm.at[pl.ds(base, B_PER_TILE)], uidx_v)
        pltpu.sync_copy(aid_hbm.at[pl.ds(base, B_PER_TILE)], aidx_v)
        pltpu.sync_copy(w_hbm, w_v)
        pltpu.sync_copy(b_hbm, b_v)

        sems = [sem0, sem1]

        def fire(h, buf):
            sem = sems[buf]
            for c in range(PASS_ROWS // LANES):
                uv = uidx_v[pl.ds(h * PASS_ROWS + c * LANES, LANES)]
                av = aidx_v[pl.ds(h * PASS_ROWS + c * LANES, LANES)]
                for l in range(LANES):
                    r = c * LANES + l
                    pltpu.async_copy(
                        ut_hbm.at[pl.ds(uv[l], 1), :],
                        ubuf_v.at[buf, pl.ds(r, 1), :], sem)
                    pltpu.async_copy(
                        at_hbm.at[pl.ds(av[l], 1), :],
                        abuf_v.at[buf, pl.ds(r, 1), :], sem)

        def drain(buf):
            pltpu.make_async_copy(
                ut_hbm.at[pl.ds(0, PASS_ROWS), :],
                ubuf_v.at[buf], sems[buf]).wait()
            pltpu.make_async_copy(
                at_hbm.at[pl.ds(0, PASS_ROWS), :],
                abuf_v.at[buf], sems[buf]).wait()

        w = w_v[...]
        b = b_v[...]
        lane = lax.iota(jnp.int32, LANES)

        def compute(h, buf):
            for grp in range(PASS_ROWS // LANES):
                off = h * PASS_ROWS + grp * LANES
                rows = lane + grp * LANES
                acc = jnp.zeros((LANES,), jnp.float32)
                for j in range(EMBED):
                    cols = (lane + j) & (EMBED - 1)
                    u = plsc.load_gather(ubuf_v.at[buf], [rows, cols])
                    a = plsc.load_gather(abuf_v.at[buf], [rows, cols])
                    acc = acc + u * a
                z = acc * w + b
                out_v[pl.ds(off, LANES)] = 1.0 / (1.0 + jnp.exp(-z))

        fire(0, 0)

        @pl.loop(0, N_PASS // 2)
        def _(i):
            p0 = 2 * i
            fire(p0 + 1, 1)
            drain(0)
            compute(p0, 0)

            @pl.when(p0 + 2 < N_PASS)
            def _():
                fire(p0 + 2, 0)

            drain(1)
            compute(p0 + 1, 1)

        pltpu.sync_copy(out_v, out_hbm.at[pl.ds(base, B_PER_TILE)])

    return kern(user_table, ad_table, user_ids, ad_ids, w_vec, b_vec)


@jax.jit
def kernel(user_ids, ad_ids, user_table, ad_table, fc_w, fc_b):
    w_vec = jnp.broadcast_to(fc_w.reshape(()), (LANES,)).astype(jnp.float32)
    b_vec = jnp.broadcast_to(fc_b.reshape(()), (LANES,)).astype(jnp.float32)
    out = _sc_kernel(user_table, ad_table, user_ids.astype(jnp.int32),
                     ad_ids.astype(jnp.int32), w_vec, b_vec)
    return out.reshape(BATCH, 1)
